# baseline (speedup 1.0000x reference)
import functools

import jax
import jax.numpy as jnp
from jax.experimental import pallas as pl
from jax.experimental.pallas import tpu as pltpu

_SLOPE_ATT = 0.2
_SLOPE_MLP = 0.01


def _lrelu(v, slope):
    return jnp.where(v > 0, v, slope * v)


def _ln(v, g, b):
    mu = jnp.mean(v, axis=-1, keepdims=True)
    var = jnp.mean(jnp.square(v - mu), axis=-1, keepdims=True)
    return (v - mu) * jax.lax.rsqrt(var + 1e-5) * g + b


def _dot(a, b):
    return jax.lax.dot_general(a, b, (((1,), (0,)), ((), ())),
                               preferred_element_type=jnp.float32)


def _dot_t(a, b):
    return jax.lax.dot_general(a, b, (((0,), (0,)), ((), ())),
                               preferred_element_type=jnp.float32)


def _dot_rr(a, b):
    return jax.lax.dot_general(a, b, (((1,), (1,)), ((), ())),
                               preferred_element_type=jnp.float32)


def _layer_kernel(x_ref, H_ref, W_ref, W2_ref, W3_ref, ahi_ref, wc_ref,
                  alo_ref, a2lo_ref, a2hi_ref,
                  hmW_ref, hmb_ref, lng_ref, lnb_ref,
                  fW1_ref, fb1_ref, fW2_ref, fb2_ref, flng_ref, flnb_ref,
                  out_ref,
                  nacc_ref, edge_ref, pr_ref, xs_ref, me_ref, sxt_ref,
                  sm_ref, *, heads, nblk, n_nodes):
    s = pl.program_id(0)
    j = jax.lax.rem(s, nblk)
    hid = W_ref.shape[2]
    nb = x_ref.shape[0]

    @pl.when(s < nblk)
    def _phase_a():
        xb = x_ref[...]
        xbb = xb.astype(jnp.bfloat16)
        ys = []
        scales = []
        for h in range(heads):
            xt = _dot(xbb, W_ref[h])
            x4 = _dot(xbb, W2_ref[h])
            c = jnp.sum(wc_ref[h] * alo_ref[h])
            s1r = _dot_rr(ahi_ref[h], x4) + c
            e1r = _lrelu(s1r, _SLOPE_ATT)
            bm = jnp.max(e1r)
            m_old = sm_ref[h]
            m_new = jnp.where(j == 0, bm, jnp.maximum(m_old, bm))
            sm_ref[h] = m_new
            scales.append(jnp.where(j == 0, 1.0, jnp.exp(m_old - m_new)))
            s1c = _dot(x4, ahi_ref[h][0][:, None]) + c
            w = jnp.exp(_lrelu(s1c, _SLOPE_ATT) - m_new)
            ys.append((w * xt).astype(jnp.bfloat16))
            ys.append(w.astype(jnp.bfloat16))
            xs = _dot_rr(a2lo_ref[h], x4)
            bx = jnp.max(xs)
            mx_old = sm_ref[heads + h]
            sm_ref[heads + h] = jnp.where(j == 0, bx,
                                          jnp.maximum(mx_old, bx))
            xs_ref[h, 0:1, pl.ds(j * nb, nb)] = xs
            sxt = jnp.sum(xt, axis=0, keepdims=True)

            @pl.when(j == 0)
            def _():
                sxt_ref[h] = sxt

            @pl.when(j > 0)
            def _():
                sxt_ref[h] = sxt_ref[h] + sxt

        pad = nacc_ref.shape[1] - (hid + 1) * heads
        Y = jnp.concatenate(
            [ys[0], ys[2], ys[1], ys[3],
             jnp.zeros((nb, pad), jnp.bfloat16)], axis=1)
        C = _dot(H_ref[...], Y)

        @pl.when(j == 0)
        def _():
            nacc_ref[...] = C

        @pl.when(j > 0)
        def _():
            need = (scales[0] < 1.0) | (scales[1] < 1.0)

            @pl.when(need)
            def _():
                li = jax.lax.broadcasted_iota(jnp.int32,
                                              (1, nacc_ref.shape[1]), 1)
                row = jnp.where(li < hid, scales[0],
                                jnp.where(li < 2 * hid, scales[1],
                                          jnp.where(li == 2 * hid, scales[0],
                                                    scales[1])))
                nacc_ref[...] = nacc_ref[...] * row

            nacc_ref[...] = nacc_ref[...] + C

    @pl.when(s == nblk)
    def _finalize():
        for h in range(heads):
            num = nacc_ref[:, h * hid:(h + 1) * hid]
            den = nacc_ref[:, 2 * hid + h:2 * hid + h + 1]
            mean_xt = sxt_ref[h] / float(n_nodes)
            edge = jnp.where(den > 0, num / jnp.where(den > 0, den, 1.0),
                             mean_xt)
            pad = edge_ref.shape[2] - hid - 1
            edge_ref[h] = jnp.concatenate(
                [edge, jnp.ones((edge.shape[0], 1), jnp.float32),
                 jnp.zeros((edge.shape[0], pad), jnp.float32)],
                axis=1).astype(jnp.bfloat16)
            e4 = _dot(edge, W3_ref[h])
            esr = _dot_rr(a2hi_ref[h], e4)
            esc = _dot(e4, a2hi_ref[h][0][:, None])
            Me = jnp.max(esr)
            Mx = sm_ref[heads + h]
            U = Me + Mx
            c1 = jnp.where(U >= 0, 1.0, jnp.exp(0.8 * U))
            c2 = jnp.where(U >= 0, jnp.exp(-0.8 * U), 1.0)
            p = jnp.exp(esc - Me) * c1
            r = jnp.exp(_SLOPE_ATT * (esc - Me)) * c2
            pr_ref[h] = jnp.concatenate([p, r], axis=1).astype(jnp.bfloat16)
            me_ref[h] = jnp.mean(edge, axis=0, keepdims=True)

    @pl.when(s >= nblk)
    def _phase_b():
        Hb = H_ref[...]
        xb = x_ref[...]
        hs = []
        for h in range(heads):
            xs = xs_ref[h, 0:1, pl.ds(j * nb, nb)]
            Mx = sm_ref[heads + h]
            q = jnp.exp(xs - Mx).astype(jnp.bfloat16)
            t = jnp.exp(_SLOPE_ATT * (xs - Mx)).astype(jnp.bfloat16)
            p = pr_ref[h][:, 0:1]
            r = pr_ref[h][:, 1:2]
            T = Hb * jnp.maximum(p * q, r * t)
            ne = _dot_t(T, edge_ref[h])
            num = ne[:, :hid]
            dcol = ne[:, hid:hid + 1]
            node = jnp.where(dcol > 0, num / jnp.where(dcol > 0, dcol, 1.0),
                             me_ref[h])
            hs.append(jnp.where(node > 0, node, jnp.exp(node) - 1.0))
        hcat = jnp.concatenate(hs, axis=-1)
        x1 = _lrelu(_dot(hcat, hmW_ref[...]) + hmb_ref[...], _SLOPE_MLP) + xb
        x1 = _ln(x1, lng_ref[...], lnb_ref[...])
        f = _lrelu(_dot(x1, fW1_ref[...]) + fb1_ref[...], _SLOPE_MLP)
        f = _lrelu(_dot(f, fW2_ref[...]) + fb2_ref[...], _SLOPE_MLP)
        f = _ln(f, flng_ref[...], flnb_ref[...])
        x2 = _ln(f + x1, lng_ref[...], lnb_ref[...])
        out_ref[...] = x2 + xb


def _layer(xb, Hb, bp, *, nb):
    n_nodes, n_in = xb.shape
    n_edges = Hb.shape[0]
    heads = len(bp['heads'])
    hid = bp['heads'][0]['W'].shape[1]
    nblk = n_nodes // nb
    nw = (hid + 1) * heads
    nw = ((nw + 127) // 128) * 128

    W_s = jnp.stack([hp['W'] for hp in bp['heads']]).astype(jnp.bfloat16)
    W2_s = jnp.stack([hp['W2'] for hp in bp['heads']]).astype(jnp.bfloat16)
    W3_s = jnp.stack([hp['W3'] for hp in bp['heads']])
    ahi_s = jnp.stack([hp['a'][hid:, 0][None, :] for hp in bp['heads']])
    wc_s = jnp.stack([hp['wc'][None, :] for hp in bp['heads']])
    alo_s = jnp.stack([hp['a'][:hid, 0][None, :] for hp in bp['heads']])
    a2lo_s = jnp.stack([hp['a2'][:hid, 0][None, :] for hp in bp['heads']])
    a2hi_s = jnp.stack([hp['a2'][hid:, 0][None, :] for hp in bp['heads']])

    f32 = jnp.float32
    full = lambda *shape: pl.BlockSpec(shape, lambda s: (0,) * len(shape))
    out = pl.pallas_call(
        functools.partial(_layer_kernel, heads=heads, nblk=nblk,
                          n_nodes=n_nodes),
        grid=(2 * nblk,),
        in_specs=[
            pl.BlockSpec((nb, n_in), lambda s: (s % nblk, 0)),
            pl.BlockSpec((n_edges, nb), lambda s: (0, s % nblk)),
            full(heads, n_in, hid),
            full(heads, n_in, hid),
            full(heads, hid, hid),
            full(heads, 1, hid), full(heads, 1, hid), full(heads, 1, hid),
            full(heads, 1, hid), full(heads, 1, hid),
            full(n_in, n_in),
            full(1, n_in), full(1, n_in), full(1, n_in),
            full(n_in, n_in), full(1, n_in),
            full(n_in, n_in), full(1, n_in),
            full(1, n_in), full(1, n_in),
        ],
        out_specs=pl.BlockSpec(
            (nb, n_in), lambda s: (jnp.where(s < nblk, 0, s % nblk), 0)),
        out_shape=jax.ShapeDtypeStruct((n_nodes, n_in), f32),
        scratch_shapes=[
            pltpu.VMEM((n_edges, nw), f32),
            pltpu.VMEM((heads, n_edges, 2 * hid), jnp.bfloat16),
            pltpu.VMEM((heads, n_edges, 2), jnp.bfloat16),
            pltpu.VMEM((heads, 1, n_nodes), f32),
            pltpu.VMEM((heads, 1, hid), f32),
            pltpu.VMEM((heads, 1, hid), f32),
            pltpu.SMEM((2 * heads,), f32),
        ],
        compiler_params=pltpu.CompilerParams(
            dimension_semantics=("arbitrary",)),
    )(xb, Hb, W_s, W2_s, W3_s, ahi_s, wc_s, alo_s, a2lo_s, a2hi_s,
      bp['hm_W'], bp['hm_b'][None, :], bp['ln_g'][None, :],
      bp['ln_b'][None, :], bp['ffn_W1'], bp['ffn_b1'][None, :],
      bp['ffn_W2'], bp['ffn_b2'][None, :], bp['ffn_ln_g'][None, :],
      bp['ffn_ln_b'][None, :])
    return out


def kernel(x, H, params):
    xb = x[0]
    Hb = H[0].astype(jnp.bfloat16)
    for bp in params:
        xb = _layer(xb, Hb, bp, nb=1024)
    return xb[None]


# speedup vs baseline: 1.0538x; 1.0538x over previous
import functools

import jax
import jax.numpy as jnp
from jax.experimental import pallas as pl
from jax.experimental.pallas import tpu as pltpu

_SLOPE_ATT = 0.2
_SLOPE_MLP = 0.01


def _lrelu(v, slope):
    return jnp.where(v > 0, v, slope * v)


def _ln(v, g, b):
    mu = jnp.mean(v, axis=-1, keepdims=True)
    var = jnp.mean(jnp.square(v - mu), axis=-1, keepdims=True)
    return (v - mu) * jax.lax.rsqrt(var + 1e-5) * g + b


def _dot(a, b):
    return jax.lax.dot_general(a, b, (((1,), (0,)), ((), ())),
                               preferred_element_type=jnp.float32)


def _dot_t(a, b):
    return jax.lax.dot_general(a, b, (((0,), (0,)), ((), ())),
                               preferred_element_type=jnp.float32)


def _dot_rr(a, b):
    return jax.lax.dot_general(a, b, (((1,), (1,)), ((), ())),
                               preferred_element_type=jnp.float32)


def _net_kernel(x_ref, H_ref, W_ref, W2_ref, W3_ref, ahi_ref, wc_ref,
                alo_ref, a2lo_ref, a2hi_ref,
                hmW_ref, hmb_ref, lng_ref, lnb_ref,
                fW1_ref, fb1_ref, fW2_ref, fb2_ref, flng_ref, flnb_ref,
                out_ref,
                nacc_ref, edge_ref, pr_ref, xs_ref, me_ref, sxt_ref,
                hsc_ref, xsc_ref, sm_ref, *, heads, nblk, n_nodes):
    s = pl.program_id(0)
    j = jax.lax.rem(s, nblk)
    hid = W_ref.shape[3]
    nb = x_ref.shape[0]
    rows = pl.ds(j * nb, nb)

    def phase_a(xb, Hbb):
        xbb = xb.astype(jnp.bfloat16)
        ys = []
        scales = []
        for h in range(heads):
            xt = _dot(xbb, W_ref[0, h])
            x4 = _dot(xbb, W2_ref[0, h])
            c = jnp.sum(wc_ref[0, h] * alo_ref[0, h])
            s1r = _dot_rr(ahi_ref[0, h], x4) + c
            e1r = _lrelu(s1r, _SLOPE_ATT)
            bm = jnp.max(e1r)
            m_old = sm_ref[h]
            m_new = jnp.where(j == 0, bm, jnp.maximum(m_old, bm))
            sm_ref[h] = m_new
            scales.append(jnp.where(j == 0, 1.0, jnp.exp(m_old - m_new)))
            s1c = _dot(x4, ahi_ref[0, h][0][:, None]) + c
            w = jnp.exp(_lrelu(s1c, _SLOPE_ATT) - m_new)
            ys.append((w * xt).astype(jnp.bfloat16))
            ys.append(w.astype(jnp.bfloat16))
            xs = _dot_rr(a2lo_ref[0, h], x4)
            bx = jnp.max(xs)
            mx_old = sm_ref[heads + h]
            sm_ref[heads + h] = jnp.where(j == 0, bx,
                                          jnp.maximum(mx_old, bx))
            xs_ref[h, 0:1, rows] = xs
            sxt = jnp.sum(xt, axis=0, keepdims=True)

            @pl.when(j == 0)
            def _():
                sxt_ref[h] = sxt

            @pl.when(j > 0)
            def _():
                sxt_ref[h] = sxt_ref[h] + sxt

        pad = nacc_ref.shape[1] - (hid + 1) * heads
        Y = jnp.concatenate(
            [ys[0], ys[2], ys[1], ys[3],
             jnp.zeros((nb, pad), jnp.bfloat16)], axis=1)
        C = _dot(Hbb, Y)

        @pl.when(j == 0)
        def _():
            nacc_ref[...] = C

        @pl.when(j > 0)
        def _():
            need = (scales[0] < 1.0) | (scales[1] < 1.0)

            @pl.when(need)
            def _():
                li = jax.lax.broadcasted_iota(jnp.int32,
                                              (1, nacc_ref.shape[1]), 1)
                row = jnp.where(li < hid, scales[0],
                                jnp.where(li < 2 * hid, scales[1],
                                          jnp.where(li == 2 * hid, scales[0],
                                                    scales[1])))
                nacc_ref[...] = nacc_ref[...] * row

            nacc_ref[...] = nacc_ref[...] + C

    def finalize():
        for h in range(heads):
            num = nacc_ref[:, h * hid:(h + 1) * hid]
            den = nacc_ref[:, 2 * hid + h:2 * hid + h + 1]
            mean_xt = sxt_ref[h] / float(n_nodes)
            edge = jnp.where(den > 0, num / jnp.where(den > 0, den, 1.0),
                             mean_xt)
            pad = edge_ref.shape[2] - hid - 1
            edge_ref[h] = jnp.concatenate(
                [edge, jnp.ones((edge.shape[0], 1), jnp.float32),
                 jnp.zeros((edge.shape[0], pad), jnp.float32)],
                axis=1).astype(jnp.bfloat16)
            e4 = _dot(edge, W3_ref[0, h])
            esr = _dot_rr(a2hi_ref[0, h], e4)
            esc = _dot(e4, a2hi_ref[0, h][0][:, None])
            Me = jnp.max(esr)
            Mx = sm_ref[heads + h]
            U = Me + Mx
            c1 = jnp.where(U >= 0, 1.0, jnp.exp(0.8 * U))
            c2 = jnp.where(U >= 0, jnp.exp(-0.8 * U), 1.0)
            p = jnp.exp(esc - Me) * c1
            r = jnp.exp(_SLOPE_ATT * (esc - Me)) * c2
            pr_ref[h] = jnp.concatenate([p, r], axis=1).astype(jnp.bfloat16)
            me_ref[h] = jnp.mean(edge, axis=0, keepdims=True)

    def phase_b(xb, Hbb):
        hs = []
        for h in range(heads):
            xs = xs_ref[h, 0:1, rows]
            Mx = sm_ref[heads + h]
            q = jnp.exp(xs - Mx).astype(jnp.bfloat16)
            t = jnp.exp(_SLOPE_ATT * (xs - Mx)).astype(jnp.bfloat16)
            p = pr_ref[h][:, 0:1]
            r = pr_ref[h][:, 1:2]
            T = Hbb * jnp.maximum(p * q, r * t)
            ne = _dot_t(T, edge_ref[h])
            num = ne[:, :hid]
            dcol = ne[:, hid:hid + 1]
            node = jnp.where(dcol > 0, num / jnp.where(dcol > 0, dcol, 1.0),
                             me_ref[h])
            hs.append(jnp.where(node > 0, node, jnp.exp(node) - 1.0))
        hcat = jnp.concatenate(hs, axis=-1)
        x1 = _lrelu(_dot(hcat, hmW_ref[0]) + hmb_ref[0], _SLOPE_MLP) + xb
        x1 = _ln(x1, lng_ref[0], lnb_ref[0])
        f = _lrelu(_dot(x1, fW1_ref[0]) + fb1_ref[0], _SLOPE_MLP)
        f = _lrelu(_dot(f, fW2_ref[0]) + fb2_ref[0], _SLOPE_MLP)
        f = _ln(f, flng_ref[0], flnb_ref[0])
        x2 = _ln(f + x1, lng_ref[0], lnb_ref[0])
        return x2 + xb

    half = 2 * nblk

    @pl.when(s < nblk)
    def _():
        xb = x_ref[...]
        Hbb = H_ref[...].astype(jnp.bfloat16)
        hsc_ref[:, rows] = Hbb
        xsc_ref[rows, :] = xb
        phase_a(xb, Hbb)

    @pl.when((s >= half) & (jax.lax.rem(s, half) < nblk))
    def _():
        phase_a(xsc_ref[rows, :], hsc_ref[:, rows])

    @pl.when(jax.lax.rem(s, half) == nblk)
    def _():
        finalize()

    @pl.when((s >= nblk) & (s < half))
    def _():
        xnew = phase_b(xsc_ref[rows, :], hsc_ref[:, rows])
        xsc_ref[rows, :] = xnew

    @pl.when(s >= half + nblk)
    def _():
        out_ref[...] = phase_b(xsc_ref[rows, :], hsc_ref[:, rows])


def kernel(x, H, params):
    xb = x[0]
    Hm = H[0]
    n_nodes, n_in = xb.shape
    n_edges = Hm.shape[0]
    heads = len(params[0]['heads'])
    hid = params[0]['heads'][0]['W'].shape[1]
    nb = 1024
    nblk = n_nodes // nb
    layers = len(params)
    nw = ((heads * (hid + 1) + 127) // 128) * 128

    bf16 = jnp.bfloat16
    f32 = jnp.float32

    def stk(f, dtype=f32):
        return jnp.stack([jnp.stack([f(hp) for hp in bp['heads']])
                          for bp in params]).astype(dtype)

    W_s = stk(lambda hp: hp['W'], bf16)
    W2_s = stk(lambda hp: hp['W2'], bf16)
    W3_s = stk(lambda hp: hp['W3'])
    ahi_s = stk(lambda hp: hp['a'][hid:, 0][None, :])
    wc_s = stk(lambda hp: hp['wc'][None, :])
    alo_s = stk(lambda hp: hp['a'][:hid, 0][None, :])
    a2lo_s = stk(lambda hp: hp['a2'][:hid, 0][None, :])
    a2hi_s = stk(lambda hp: hp['a2'][hid:, 0][None, :])
    hmW_s = jnp.stack([bp['hm_W'] for bp in params])
    hmb_s = jnp.stack([bp['hm_b'][None, :] for bp in params])
    lng_s = jnp.stack([bp['ln_g'][None, :] for bp in params])
    lnb_s = jnp.stack([bp['ln_b'][None, :] for bp in params])
    fW1_s = jnp.stack([bp['ffn_W1'] for bp in params])
    fb1_s = jnp.stack([bp['ffn_b1'][None, :] for bp in params])
    fW2_s = jnp.stack([bp['ffn_W2'] for bp in params])
    fb2_s = jnp.stack([bp['ffn_b2'][None, :] for bp in params])
    flng_s = jnp.stack([bp['ffn_ln_g'][None, :] for bp in params])
    flnb_s = jnp.stack([bp['ffn_ln_b'][None, :] for bp in params])

    half = 2 * nblk

    def lfull(extra):
        return pl.BlockSpec((1,) + extra,
                            lambda s: (s // half,) + (0,) * len(extra))

    out = pl.pallas_call(
        functools.partial(_net_kernel, heads=heads, nblk=nblk,
                          n_nodes=n_nodes),
        grid=(layers * half,),
        in_specs=[
            pl.BlockSpec((nb, n_in),
                         lambda s: (jnp.where(s < nblk, s, nblk - 1), 0)),
            pl.BlockSpec((n_edges, nb),
                         lambda s: (0, jnp.where(s < nblk, s, nblk - 1))),
            lfull((heads, n_in, hid)),
            lfull((heads, n_in, hid)),
            lfull((heads, hid, hid)),
            lfull((heads, 1, hid)), lfull((heads, 1, hid)),
            lfull((heads, 1, hid)), lfull((heads, 1, hid)),
            lfull((heads, 1, hid)),
            lfull((n_in, n_in)),
            lfull((1, n_in)), lfull((1, n_in)), lfull((1, n_in)),
            lfull((n_in, n_in)), lfull((1, n_in)),
            lfull((n_in, n_in)), lfull((1, n_in)),
            lfull((1, n_in)), lfull((1, n_in)),
        ],
        out_specs=pl.BlockSpec(
            (nb, n_in),
            lambda s: (jnp.where(s >= (2 * layers - 1) * nblk,
                                 s % nblk, 0), 0)),
        out_shape=jax.ShapeDtypeStruct((n_nodes, n_in), f32),
        scratch_shapes=[
            pltpu.VMEM((n_edges, nw), f32),
            pltpu.VMEM((heads, n_edges, 2 * hid), bf16),
            pltpu.VMEM((heads, n_edges, 2), bf16),
            pltpu.VMEM((heads, 1, n_nodes), f32),
            pltpu.VMEM((heads, 1, hid), f32),
            pltpu.VMEM((heads, 1, hid), f32),
            pltpu.VMEM((n_edges, n_nodes), bf16),
            pltpu.VMEM((n_nodes, n_in), f32),
            pltpu.SMEM((2 * heads,), f32),
        ],
        compiler_params=pltpu.CompilerParams(
            dimension_semantics=("arbitrary",)),
    )(xb, Hm, W_s, W2_s, W3_s, ahi_s, wc_s, alo_s, a2lo_s, a2hi_s,
      hmW_s, hmb_s, lng_s, lnb_s, fW1_s, fb1_s, fW2_s, fb2_s,
      flng_s, flnb_s)
    return out[None]


# speedup vs baseline: 1.1704x; 1.1107x over previous
import functools

import jax
import jax.numpy as jnp
from jax.experimental import pallas as pl
from jax.experimental.pallas import tpu as pltpu

_SLOPE_ATT = 0.2
_SLOPE_MLP = 0.01


def _lrelu(v, slope):
    return jnp.where(v > 0, v, slope * v)


def _ln(v, g, b):
    mu = jnp.mean(v, axis=-1, keepdims=True)
    var = jnp.mean(jnp.square(v - mu), axis=-1, keepdims=True)
    return (v - mu) * jax.lax.rsqrt(var + 1e-5) * g + b


def _dot(a, b):
    return jax.lax.dot_general(a, b, (((1,), (0,)), ((), ())),
                               preferred_element_type=jnp.float32)


def _dot_t(a, b):
    return jax.lax.dot_general(a, b, (((0,), (0,)), ((), ())),
                               preferred_element_type=jnp.float32)


def _dot_rr(a, b):
    return jax.lax.dot_general(a, b, (((1,), (1,)), ((), ())),
                               preferred_element_type=jnp.float32)


def _net_kernel(x_ref, H_ref, W_ref, W2_ref, W3_ref, ahi_ref, wc_ref,
                alo_ref, a2lo_ref, a2hi_ref,
                hmW_ref, hmb_ref, lng_ref, lnb_ref,
                fW1_ref, fb1_ref, fW2_ref, fb2_ref, flng_ref, flnb_ref,
                out_ref,
                nacc_ref, edge_ref, pr_ref, xs_ref, me_ref, sxt_ref,
                hsc_ref, xsc_ref, sm_ref, *, heads, nblk, n_nodes):
    s = pl.program_id(0)
    j = jax.lax.rem(s, nblk)
    hid = W_ref.shape[3]
    nb = x_ref.shape[0]
    rows = pl.ds(j * nb, nb)

    def phase_a(xb, Hbb):
        xbb = xb.astype(jnp.bfloat16)
        ys = []
        scales = []
        for h in range(heads):
            xt = _dot(xbb, W_ref[0, h])
            x4 = _dot(xbb, W2_ref[0, h])
            c = jnp.sum(wc_ref[0, h] * alo_ref[0, h])
            s1r = _dot_rr(ahi_ref[0, h], x4) + c
            e1r = _lrelu(s1r, _SLOPE_ATT)
            bm = jnp.max(e1r)
            m_old = sm_ref[h]
            m_new = jnp.where(j == 0, bm, jnp.maximum(m_old, bm))
            sm_ref[h] = m_new
            scales.append(jnp.where(j == 0, 1.0, jnp.exp(m_old - m_new)))
            s1c = _dot(x4, ahi_ref[0, h][0][:, None]) + c
            w = jnp.exp(_lrelu(s1c, _SLOPE_ATT) - m_new)
            ys.append((w * xt).astype(jnp.bfloat16))
            ys.append(w.astype(jnp.bfloat16))
            xs = _dot_rr(a2lo_ref[0, h], x4)
            bx = jnp.max(xs)
            mx_old = sm_ref[heads + h]
            sm_ref[heads + h] = jnp.where(j == 0, bx,
                                          jnp.maximum(mx_old, bx))
            xs_ref[h, 0:1, rows] = xs
            sxt = jnp.sum(xt, axis=0, keepdims=True)

            @pl.when(j == 0)
            def _():
                sxt_ref[h] = sxt

            @pl.when(j > 0)
            def _():
                sxt_ref[h] = sxt_ref[h] + sxt

        pad = nacc_ref.shape[1] - (hid + 1) * heads
        Y = jnp.concatenate(
            [ys[0], ys[2], ys[1], ys[3],
             jnp.zeros((nb, pad), jnp.bfloat16)], axis=1)
        C = _dot(Hbb, Y)

        @pl.when(j == 0)
        def _():
            nacc_ref[...] = C

        @pl.when(j > 0)
        def _():
            need = (scales[0] < 1.0) | (scales[1] < 1.0)

            @pl.when(need)
            def _():
                li = jax.lax.broadcasted_iota(jnp.int32,
                                              (1, nacc_ref.shape[1]), 1)
                row = jnp.where(li < hid, scales[0],
                                jnp.where(li < 2 * hid, scales[1],
                                          jnp.where(li == 2 * hid, scales[0],
                                                    scales[1])))
                nacc_ref[...] = nacc_ref[...] * row

            nacc_ref[...] = nacc_ref[...] + C

    def finalize():
        for h in range(heads):
            num = nacc_ref[:, h * hid:(h + 1) * hid]
            den = nacc_ref[:, 2 * hid + h:2 * hid + h + 1]
            mean_xt = sxt_ref[h] / float(n_nodes)
            edge = jnp.where(den > 0, num / jnp.where(den > 0, den, 1.0),
                             mean_xt)
            pad = edge_ref.shape[2] - hid - 1
            edge_ref[h] = jnp.concatenate(
                [edge, jnp.ones((edge.shape[0], 1), jnp.float32),
                 jnp.zeros((edge.shape[0], pad), jnp.float32)],
                axis=1).astype(jnp.bfloat16)
            e4 = _dot(edge, W3_ref[0, h])
            esr = _dot_rr(a2hi_ref[0, h], e4)
            esc = _dot(e4, a2hi_ref[0, h][0][:, None])
            Me = jnp.max(esr)
            Mx = sm_ref[heads + h]
            U = Me + Mx
            c1 = jnp.where(U >= 0, 1.0, jnp.exp(0.8 * U))
            c2 = jnp.where(U >= 0, jnp.exp(-0.8 * U), 1.0)
            p = jnp.exp(esc - Me) * c1
            r = jnp.exp(_SLOPE_ATT * (esc - Me)) * c2
            pr_ref[h] = jnp.concatenate([p, r], axis=1).astype(jnp.bfloat16)
            me_ref[h] = jnp.mean(edge, axis=0, keepdims=True)

    def phase_b(xb, Hbb):
        hs = []
        for h in range(heads):
            xs = xs_ref[h, 0:1, rows]
            Mx = sm_ref[heads + h]
            q = jnp.exp(xs - Mx).astype(jnp.bfloat16)
            t = jnp.exp(_SLOPE_ATT * (xs - Mx)).astype(jnp.bfloat16)
            p = pr_ref[h][:, 0:1]
            r = pr_ref[h][:, 1:2]
            T = Hbb * jnp.maximum(p * q, r * t)
            ne = _dot_t(T, edge_ref[h])
            num = ne[:, :hid]
            dcol = ne[:, hid:hid + 1]
            node = jnp.where(dcol > 0, num / jnp.where(dcol > 0, dcol, 1.0),
                             me_ref[h])
            hs.append(jnp.where(node > 0, node, jnp.exp(node) - 1.0))
        hcat = jnp.concatenate(hs, axis=-1)
        x1 = _lrelu(_dot(hcat, hmW_ref[0]) + hmb_ref[0], _SLOPE_MLP) + xb
        x1 = _ln(x1, lng_ref[0], lnb_ref[0])
        f = _lrelu(_dot(x1, fW1_ref[0]) + fb1_ref[0], _SLOPE_MLP)
        f = _lrelu(_dot(f, fW2_ref[0]) + fb2_ref[0], _SLOPE_MLP)
        f = _ln(f, flng_ref[0], flnb_ref[0])
        x2 = _ln(f + x1, lng_ref[0], lnb_ref[0])
        return x2 + xb

    half = 2 * nblk

    @pl.when(s < nblk)
    def _():
        xb = x_ref[...]
        Hbb = H_ref[...].astype(jnp.bfloat16)
        hsc_ref[:, rows] = Hbb
        xsc_ref[rows, :] = xb
        phase_a(xb, Hbb)

    @pl.when((s >= half) & (jax.lax.rem(s, half) < nblk))
    def _():
        phase_a(xsc_ref[rows, :], hsc_ref[:, rows])

    @pl.when(jax.lax.rem(s, half) == nblk)
    def _():
        finalize()

    @pl.when((s >= nblk) & (s < half))
    def _():
        xnew = phase_b(xsc_ref[rows, :], hsc_ref[:, rows])
        xsc_ref[rows, :] = xnew

    @pl.when(s >= half + nblk)
    def _():
        out_ref[...] = phase_b(xsc_ref[rows, :], hsc_ref[:, rows])


def kernel(x, H, params):
    xb = x[0]
    Hm = H[0]
    n_nodes, n_in = xb.shape
    n_edges = Hm.shape[0]
    heads = len(params[0]['heads'])
    hid = params[0]['heads'][0]['W'].shape[1]
    nb = 2048
    nblk = n_nodes // nb
    layers = len(params)
    nw = ((heads * (hid + 1) + 127) // 128) * 128

    bf16 = jnp.bfloat16
    f32 = jnp.float32

    def stk(f, dtype=f32):
        return jnp.stack([jnp.stack([f(hp) for hp in bp['heads']])
                          for bp in params]).astype(dtype)

    W_s = stk(lambda hp: hp['W'], bf16)
    W2_s = stk(lambda hp: hp['W2'], bf16)
    W3_s = stk(lambda hp: hp['W3'])
    ahi_s = stk(lambda hp: hp['a'][hid:, 0][None, :])
    wc_s = stk(lambda hp: hp['wc'][None, :])
    alo_s = stk(lambda hp: hp['a'][:hid, 0][None, :])
    a2lo_s = stk(lambda hp: hp['a2'][:hid, 0][None, :])
    a2hi_s = stk(lambda hp: hp['a2'][hid:, 0][None, :])
    hmW_s = jnp.stack([bp['hm_W'] for bp in params])
    hmb_s = jnp.stack([bp['hm_b'][None, :] for bp in params])
    lng_s = jnp.stack([bp['ln_g'][None, :] for bp in params])
    lnb_s = jnp.stack([bp['ln_b'][None, :] for bp in params])
    fW1_s = jnp.stack([bp['ffn_W1'] for bp in params])
    fb1_s = jnp.stack([bp['ffn_b1'][None, :] for bp in params])
    fW2_s = jnp.stack([bp['ffn_W2'] for bp in params])
    fb2_s = jnp.stack([bp['ffn_b2'][None, :] for bp in params])
    flng_s = jnp.stack([bp['ffn_ln_g'][None, :] for bp in params])
    flnb_s = jnp.stack([bp['ffn_ln_b'][None, :] for bp in params])

    half = 2 * nblk

    def lfull(extra):
        return pl.BlockSpec((1,) + extra,
                            lambda s: (s // half,) + (0,) * len(extra))

    out = pl.pallas_call(
        functools.partial(_net_kernel, heads=heads, nblk=nblk,
                          n_nodes=n_nodes),
        grid=(layers * half,),
        in_specs=[
            pl.BlockSpec((nb, n_in),
                         lambda s: (jnp.where(s < nblk, s, nblk - 1), 0)),
            pl.BlockSpec((n_edges, nb),
                         lambda s: (0, jnp.where(s < nblk, s, nblk - 1))),
            lfull((heads, n_in, hid)),
            lfull((heads, n_in, hid)),
            lfull((heads, hid, hid)),
            lfull((heads, 1, hid)), lfull((heads, 1, hid)),
            lfull((heads, 1, hid)), lfull((heads, 1, hid)),
            lfull((heads, 1, hid)),
            lfull((n_in, n_in)),
            lfull((1, n_in)), lfull((1, n_in)), lfull((1, n_in)),
            lfull((n_in, n_in)), lfull((1, n_in)),
            lfull((n_in, n_in)), lfull((1, n_in)),
            lfull((1, n_in)), lfull((1, n_in)),
        ],
        out_specs=pl.BlockSpec(
            (nb, n_in),
            lambda s: (jnp.where(s >= (2 * layers - 1) * nblk,
                                 s % nblk, 0), 0)),
        out_shape=jax.ShapeDtypeStruct((n_nodes, n_in), f32),
        scratch_shapes=[
            pltpu.VMEM((n_edges, nw), f32),
            pltpu.VMEM((heads, n_edges, 2 * hid), bf16),
            pltpu.VMEM((heads, n_edges, 2), bf16),
            pltpu.VMEM((heads, 1, n_nodes), f32),
            pltpu.VMEM((heads, 1, hid), f32),
            pltpu.VMEM((heads, 1, hid), f32),
            pltpu.VMEM((n_edges, n_nodes), bf16),
            pltpu.VMEM((n_nodes, n_in), f32),
            pltpu.SMEM((2 * heads,), f32),
        ],
        compiler_params=pltpu.CompilerParams(
            dimension_semantics=("arbitrary",)),
    )(xb, Hm, W_s, W2_s, W3_s, ahi_s, wc_s, alo_s, a2lo_s, a2hi_s,
      hmW_s, hmb_s, lng_s, lnb_s, fW1_s, fb1_s, fW2_s, fb2_s,
      flng_s, flnb_s)
    return out[None]
